# baseline (speedup 1.0000x reference)
import jax
import jax.numpy as jnp
from jax.experimental import pallas as pl
from jax.experimental.pallas import tpu as pltpu

NEG_INF = float("-inf")
BLK = 80
OUT = 10


def _nms_one(x):
    row = jax.lax.broadcasted_iota(jnp.int32, x.shape, 1)
    h = x.shape[1]
    left = jnp.concatenate([jnp.full_like(x[:, :, :1], NEG_INF), x[:, :, :-1]], axis=2)
    right = jnp.concatenate([x[:, :, 1:], jnp.full_like(x[:, :, :1], NEG_INF)], axis=2)
    rowmax = jnp.maximum(jnp.maximum(left, x), right)
    up = jnp.where(row == 0, NEG_INF, pltpu.roll(rowmax, 1, 1))
    down = jnp.where(row == h - 1, NEG_INF, pltpu.roll(rowmax, h - 1, 1))
    hmax = jnp.maximum(jnp.maximum(up, rowmax), down)
    return jnp.where(hmax == x, x, 0.0)


def _nms_body(x_ref, o_ref):
    s = pl.program_id(1)
    for j in range(OUT):
        x = x_ref[pl.ds(s * OUT + j, 1)]
        o_ref[j : j + 1] = _nms_one(x)


def kernel(points):
    n, c, h, w = points.shape
    x = points.reshape(n * c, h, w)
    out = pl.pallas_call(
        _nms_body,
        grid=((n * c) // BLK, BLK // OUT),
        in_specs=[pl.BlockSpec((BLK, h, w), lambda i, s: (i, 0, 0))],
        out_specs=pl.BlockSpec((OUT, h, w), lambda i, s: (i * (BLK // OUT) + s, 0, 0)),
        out_shape=jax.ShapeDtypeStruct((n * c, h, w), points.dtype),
        compiler_params=pltpu.CompilerParams(vmem_limit_bytes=128 * 1024 * 1024),
    )(x)
    return out.reshape(n, c, h, w)


# speedup vs baseline: 1.3899x; 1.3899x over previous
import jax
import jax.numpy as jnp
from jax.experimental import pallas as pl
from jax.experimental.pallas import tpu as pltpu

NEG_INF = float("-inf")
BLK = 40


def _w_pass(x):
    left = jnp.concatenate([jnp.full_like(x[:, :, :1], NEG_INF), x[:, :, :-1]], axis=2)
    right = jnp.concatenate([x[:, :, 1:], jnp.full_like(x[:, :, :1], NEG_INF)], axis=2)
    return jnp.maximum(jnp.maximum(left, x), right)


def _h_pass(x, rowmax):
    row = jax.lax.broadcasted_iota(jnp.int32, x.shape, 1)
    h = x.shape[1]
    up = jnp.where(row == 0, NEG_INF, pltpu.roll(rowmax, 1, 1))
    down = jnp.where(row == h - 1, NEG_INF, pltpu.roll(rowmax, h - 1, 1))
    hmax = jnp.maximum(jnp.maximum(up, rowmax), down)
    return jnp.where(hmax == x, x, 0.0)


def _nms_body(x_ref, o_ref):
    prev = None
    for j in range(BLK):
        x = x_ref[j : j + 1]
        rm = _w_pass(x)
        if prev is not None:
            pj, px, prm = prev
            o_ref[pj : pj + 1] = _h_pass(px, prm)
        prev = (j, x, rm)
    pj, px, prm = prev
    o_ref[pj : pj + 1] = _h_pass(px, prm)


def kernel(points):
    n, c, h, w = points.shape
    x = points.reshape(n * c, h, w)
    out = pl.pallas_call(
        _nms_body,
        grid=((n * c) // BLK,),
        in_specs=[pl.BlockSpec((BLK, h, w), lambda i: (i, 0, 0))],
        out_specs=pl.BlockSpec((BLK, h, w), lambda i: (i, 0, 0)),
        out_shape=jax.ShapeDtypeStruct((n * c, h, w), points.dtype),
        compiler_params=pltpu.CompilerParams(vmem_limit_bytes=128 * 1024 * 1024),
    )(x)
    return out.reshape(n, c, h, w)
